# Initial kernel scaffold; baseline (speedup 1.0000x reference)
import jax
import jax.numpy as jnp
from jax.experimental import pallas as pl


def kernel(edge_feats, node_feats, edge_index, W_edge, W_src, W_dst, b1, W_out, b_out):
    raise NotImplementedError("write your pallas kernel here")


# speedup vs baseline: 2.1857x; 2.1857x over previous
import functools

import jax
import jax.numpy as jnp
from jax import lax
from jax.experimental import pallas as pl
from jax.experimental.pallas import tpu as pltpu
from jax.experimental.pallas import tpu_sc as plsc

N_NODES = 10000
N_EDGES = 320000
D = 128

NC = 2
NS = 16
NW = NC * NS
EDGES_PER_WORKER = N_EDGES // NW
CHUNK = 80
NCHUNKS = EDGES_PER_WORKER // CHUNK


def _node_proj_body(nf_ref, ws_ref, wd_ref, b1_ref, ts_ref, td_ref):
    nf = nf_ref[...]
    dn = (((1,), (1,)), ((), ()))
    ts_ref[...] = lax.dot_general(nf, ws_ref[...], dn, preferred_element_type=jnp.float32)
    td_ref[...] = (
        lax.dot_general(nf, wd_ref[...], dn, preferred_element_type=jnp.float32)
        + b1_ref[...]
    )


def _node_proj(node_feats, W_src, W_dst, b1):
    nblk = 2000
    grid = (N_NODES // nblk,)
    return pl.pallas_call(
        _node_proj_body,
        grid=grid,
        in_specs=[
            pl.BlockSpec((nblk, D), lambda i: (i, 0)),
            pl.BlockSpec((D, D), lambda i: (0, 0)),
            pl.BlockSpec((D, D), lambda i: (0, 0)),
            pl.BlockSpec((1, D), lambda i: (0, 0)),
        ],
        out_specs=[
            pl.BlockSpec((nblk, D), lambda i: (i, 0)),
            pl.BlockSpec((nblk, D), lambda i: (i, 0)),
        ],
        out_shape=[
            jax.ShapeDtypeStruct((N_NODES, D), jnp.float32),
            jax.ShapeDtypeStruct((N_NODES, D), jnp.float32),
        ],
    )(node_feats, W_src, W_dst, b1)


def _gather_sum_body(ts_hbm, td_hbm, src_hbm, dst_hbm, out_hbm,
                     idx_s, idx_d, rows_s, rows_d, sem_s, sem_d):
    wid = lax.axis_index("s") * NC + lax.axis_index("c")

    def chunk_body(c, carry):
        base = wid * EDGES_PER_WORKER + c * CHUNK
        pltpu.sync_copy(src_hbm.at[pl.ds(base, CHUNK)], idx_s)
        pltpu.sync_copy(dst_hbm.at[pl.ds(base, CHUNK)], idx_d)
        cp_s = pltpu.async_copy(ts_hbm.at[idx_s], rows_s, sem_s)
        cp_d = pltpu.async_copy(td_hbm.at[idx_d], rows_d, sem_d)
        cp_s.wait()
        cp_d.wait()

        def add_body(e, acc):
            for j in range(D // 16):
                sl = pl.ds(j * 16, 16)
                rows_s[e, sl] = rows_s[e, sl] + rows_d[e, sl]
            return acc

        lax.fori_loop(0, CHUNK, add_body, 0)
        pltpu.sync_copy(rows_s, out_hbm.at[pl.ds(base, CHUNK)])
        return carry

    lax.fori_loop(0, NCHUNKS, chunk_body, 0)


def _gather_sum(t_src, t_dst, src, dst):
    mesh = plsc.VectorSubcoreMesh(
        core_axis_name="c", subcore_axis_name="s", num_cores=NC, num_subcores=NS
    )
    fn = pl.kernel(
        _gather_sum_body,
        out_type=jax.ShapeDtypeStruct((N_EDGES, D), jnp.float32),
        mesh=mesh,
        scratch_types=[
            pltpu.VMEM((CHUNK,), jnp.int32),
            pltpu.VMEM((CHUNK,), jnp.int32),
            pltpu.VMEM((CHUNK, D), jnp.float32),
            pltpu.VMEM((CHUNK, D), jnp.float32),
            pltpu.SemaphoreType.DMA,
            pltpu.SemaphoreType.DMA,
        ],
    )
    return fn(t_src, t_dst, src, dst)


def _edge_mlp_body(ef_ref, g_ref, we_ref, wo_ref, bo_ref, out_ref):
    dn = (((1,), (1,)), ((), ()))
    h = lax.dot_general(ef_ref[...], we_ref[...], dn, preferred_element_type=jnp.float32)
    h = jnp.maximum(h + g_ref[...], 0.0)
    out_ref[...] = (
        lax.dot_general(h, wo_ref[...], dn, preferred_element_type=jnp.float32)
        + bo_ref[...]
    )


def _edge_mlp(edge_feats, g, W_edge, W_out, b_out):
    eblk = 1280
    grid = (N_EDGES // eblk,)
    return pl.pallas_call(
        _edge_mlp_body,
        grid=grid,
        in_specs=[
            pl.BlockSpec((eblk, D), lambda i: (i, 0)),
            pl.BlockSpec((eblk, D), lambda i: (i, 0)),
            pl.BlockSpec((D, D), lambda i: (0, 0)),
            pl.BlockSpec((D, D), lambda i: (0, 0)),
            pl.BlockSpec((1, D), lambda i: (0, 0)),
        ],
        out_specs=pl.BlockSpec((eblk, D), lambda i: (i, 0)),
        out_shape=jax.ShapeDtypeStruct((N_EDGES, D), jnp.float32),
        compiler_params=pltpu.CompilerParams(
            dimension_semantics=("arbitrary",),
        ),
    )(edge_feats, g, W_edge, W_out, b_out)


def kernel(edge_feats, node_feats, edge_index, W_edge, W_src, W_dst, b1, W_out, b_out):
    src = edge_index[0].astype(jnp.int32)
    dst = edge_index[1].astype(jnp.int32)
    t_src, t_dst = _node_proj(node_feats, W_src, W_dst, b1.reshape(1, D))
    g = _gather_sum(t_src, t_dst, src, dst)
    return _edge_mlp(edge_feats, g, W_edge, W_out, b_out.reshape(1, D))


# speedup vs baseline: 2.8300x; 1.2948x over previous
import functools

import jax
import jax.numpy as jnp
from jax import lax
from jax.experimental import pallas as pl
from jax.experimental.pallas import tpu as pltpu
from jax.experimental.pallas import tpu_sc as plsc

N_NODES = 10000
N_EDGES = 320000
D = 128

NC = 2
NS = 16
NW = NC * NS
EDGES_PER_WORKER = N_EDGES // NW
CHUNK = 80
NCHUNKS = EDGES_PER_WORKER // CHUNK


def _node_proj_body(nf_ref, ws_ref, wd_ref, b1_ref, ts_ref, td_ref):
    nf = nf_ref[...]
    dn = (((1,), (1,)), ((), ()))
    ts_ref[...] = lax.dot_general(nf, ws_ref[...], dn, preferred_element_type=jnp.float32)
    td_ref[...] = (
        lax.dot_general(nf, wd_ref[...], dn, preferred_element_type=jnp.float32)
        + b1_ref[...]
    )


def _node_proj(node_feats, W_src, W_dst, b1):
    nblk = 2000
    grid = (N_NODES // nblk,)
    return pl.pallas_call(
        _node_proj_body,
        grid=grid,
        in_specs=[
            pl.BlockSpec((nblk, D), lambda i: (i, 0)),
            pl.BlockSpec((D, D), lambda i: (0, 0)),
            pl.BlockSpec((D, D), lambda i: (0, 0)),
            pl.BlockSpec((1, D), lambda i: (0, 0)),
        ],
        out_specs=[
            pl.BlockSpec((nblk, D), lambda i: (i, 0)),
            pl.BlockSpec((nblk, D), lambda i: (i, 0)),
        ],
        out_shape=[
            jax.ShapeDtypeStruct((N_NODES, D), jnp.float32),
            jax.ShapeDtypeStruct((N_NODES, D), jnp.float32),
        ],
    )(node_feats, W_src, W_dst, b1)


def _gather_sum_body(ts_hbm, td_hbm, src_hbm, dst_hbm, out_hbm,
                     idx_s, idx_d, rows_s, rows_d, sem_s, sem_d):
    wid = lax.axis_index("s") * NC + lax.axis_index("c")
    wbase = wid * EDGES_PER_WORKER

    def issue(c, b):
        base = wbase + c * CHUNK
        pltpu.sync_copy(src_hbm.at[pl.ds(base, CHUNK)], idx_s[b])
        pltpu.sync_copy(dst_hbm.at[pl.ds(base, CHUNK)], idx_d[b])
        pltpu.async_copy(ts_hbm.at[idx_s[b]], rows_s[b], sem_s[b])
        pltpu.async_copy(td_hbm.at[idx_d[b]], rows_d[b], sem_d[b])

    def process(c, b):
        pltpu.make_async_copy(ts_hbm.at[idx_s[b]], rows_s[b], sem_s[b]).wait()
        pltpu.make_async_copy(td_hbm.at[idx_d[b]], rows_d[b], sem_d[b]).wait()

        def add_body(e, acc):
            for j in range(D // 16):
                sl = pl.ds(j * 16, 16)
                rows_s[b][e, sl] = rows_s[b][e, sl] + rows_d[b][e, sl]
            return acc

        lax.fori_loop(0, CHUNK, add_body, 0)
        pltpu.sync_copy(rows_s[b], out_hbm.at[pl.ds(wbase + c * CHUNK, CHUNK)])

    issue(0, 0)

    def pair_body(p, carry):
        for b in range(2):
            c = 2 * p + b
            nxt = c + 1

            @pl.when(nxt < NCHUNKS)
            def _():
                issue(nxt, 1 - b)

            @pl.when(c < NCHUNKS)
            def _():
                process(c, b)
        return carry

    lax.fori_loop(0, (NCHUNKS + 1) // 2, pair_body, 0)


def _gather_sum(t_src, t_dst, src, dst):
    mesh = plsc.VectorSubcoreMesh(
        core_axis_name="c", subcore_axis_name="s", num_cores=NC, num_subcores=NS
    )
    fn = pl.kernel(
        _gather_sum_body,
        out_type=jax.ShapeDtypeStruct((N_EDGES, D), jnp.float32),
        mesh=mesh,
        scratch_types=[
            [pltpu.VMEM((CHUNK,), jnp.int32) for _ in range(2)],
            [pltpu.VMEM((CHUNK,), jnp.int32) for _ in range(2)],
            [pltpu.VMEM((CHUNK, D), jnp.float32) for _ in range(2)],
            [pltpu.VMEM((CHUNK, D), jnp.float32) for _ in range(2)],
            [pltpu.SemaphoreType.DMA for _ in range(2)],
            [pltpu.SemaphoreType.DMA for _ in range(2)],
        ],
    )
    return fn(t_src, t_dst, src, dst)


def _edge_mlp_body(ef_ref, g_ref, we_ref, wo_ref, bo_ref, out_ref):
    dn = (((1,), (1,)), ((), ()))
    h = lax.dot_general(ef_ref[...], we_ref[...], dn, preferred_element_type=jnp.float32)
    h = jnp.maximum(h + g_ref[...], 0.0)
    out_ref[...] = (
        lax.dot_general(h, wo_ref[...], dn, preferred_element_type=jnp.float32)
        + bo_ref[...]
    )


def _edge_mlp(edge_feats, g, W_edge, W_out, b_out):
    eblk = 1280
    grid = (N_EDGES // eblk,)
    return pl.pallas_call(
        _edge_mlp_body,
        grid=grid,
        in_specs=[
            pl.BlockSpec((eblk, D), lambda i: (i, 0)),
            pl.BlockSpec((eblk, D), lambda i: (i, 0)),
            pl.BlockSpec((D, D), lambda i: (0, 0)),
            pl.BlockSpec((D, D), lambda i: (0, 0)),
            pl.BlockSpec((1, D), lambda i: (0, 0)),
        ],
        out_specs=pl.BlockSpec((eblk, D), lambda i: (i, 0)),
        out_shape=jax.ShapeDtypeStruct((N_EDGES, D), jnp.float32),
        compiler_params=pltpu.CompilerParams(
            dimension_semantics=("arbitrary",),
        ),
    )(edge_feats, g, W_edge, W_out, b_out)


def kernel(edge_feats, node_feats, edge_index, W_edge, W_src, W_dst, b1, W_out, b_out):
    src = edge_index[0].astype(jnp.int32)
    dst = edge_index[1].astype(jnp.int32)
    t_src, t_dst = _node_proj(node_feats, W_src, W_dst, b1.reshape(1, D))
    g = _gather_sum(t_src, t_dst, src, dst)
    return _edge_mlp(edge_feats, g, W_edge, W_out, b_out.reshape(1, D))
